# baseline (speedup 1.0000x reference)
import jax
import jax.numpy as jnp
from jax import lax
from jax.experimental import pallas as pl
from jax.experimental.pallas import tpu as pltpu
from jax.experimental.pallas import tpu_sc as plsc

N = 10000
E = 320000
ALPHA = 0.2

NC = 2
NS = 16
NW = NC * NS
EDGES_PER_TILE = E // NW
SLICE = 624
REM = N - NS * SLICE
DEGW = 8


def _leaky(v):
    return jnp.where(v >= 0, v, ALPHA * v)


def _matT(a, w):
    return lax.dot_general(a, w, (((1,), (1,)), ((), ())),
                           preferred_element_type=jnp.float32)


def _make_seg_sum(d, with_deg, K):
    CHUNKS = EDGES_PER_TILE // K
    assert EDGES_PER_TILE % K == 0 and K % 8 == 0
    SPAN = 64
    IDXW = SPAN * K
    mesh = plsc.VectorSubcoreMesh(core_axis_name="c", subcore_axis_name="s")
    out_type = [jax.ShapeDtypeStruct((NC * N, 128), jnp.float32)]
    scratch = [
        pltpu.VMEM((IDXW,), jnp.int32),
        pltpu.VMEM((IDXW,), jnp.int32),
        pltpu.VMEM((K, d), jnp.float32),
        pltpu.VMEM((K, d), jnp.float32),
        pltpu.VMEM((K, d), jnp.float32),
        pltpu.VMEM_SHARED((N, d), jnp.float32),
        pltpu.SemaphoreType.DMA,
        pltpu.SemaphoreType.DMA,
        pltpu.SemaphoreType.DMA,
        pltpu.SemaphoreType.DMA,
        pltpu.SemaphoreType.DMA,
        pltpu.SemaphoreType.DMA,
    ]
    if with_deg:
        out_type.append(jax.ShapeDtypeStruct((NC * N, 128), jnp.float32))
        scratch += [
            pltpu.VMEM((K, DEGW), jnp.float32),
            pltpu.VMEM_SHARED((N, DEGW), jnp.float32),
            pltpu.SemaphoreType.DMA,
            pltpu.SemaphoreType.DMA,
            pltpu.SemaphoreType.DMA,
        ]

    def body(h_hbm, src_hbm, dst_hbm, zd_hbm, ones_hbm, p_hbm,
             dp_hbm, src_all, dst_all, bufs, acc, gsem, ssem, ones, dacc,
             dsem):
        c = lax.axis_index("c")
        s = lax.axis_index("s")
        wid = c * NS + s
        r0 = s * SLICE

        def zrow(i, _):
            for jj in range(d // 16):
                bufs[0][i, pl.ds(16 * jj, 16)] = jnp.zeros((16,), jnp.float32)
            return ()

        lax.fori_loop(0, K, zrow, (), unroll=False)
        for t in range(SLICE // K):
            pltpu.async_copy(bufs[0], acc.at[pl.ds(r0 + t * K, K)], gsem[0])
        zrem = SLICE - (SLICE // K) * K
        if zrem:
            pltpu.async_copy(bufs[0].at[pl.ds(0, zrem)],
                             acc.at[pl.ds(r0 + SLICE - zrem, zrem)], gsem[0])

        @pl.when(s == 0)
        def _():
            pltpu.async_copy(bufs[0].at[pl.ds(0, REM)],
                             acc.at[pl.ds(NS * SLICE, REM)], gsem[0])
        for t in range(SLICE // K):
            pltpu.make_async_copy(bufs[0], acc.at[pl.ds(r0 + t * K, K)],
                                  gsem[0]).wait()
        if zrem:
            pltpu.make_async_copy(bufs[0].at[pl.ds(0, zrem)],
                                  acc.at[pl.ds(r0 + SLICE - zrem, zrem)],
                                  gsem[0]).wait()

        @pl.when(s == 0)
        def _():
            pltpu.make_async_copy(bufs[0].at[pl.ds(0, REM)],
                                  acc.at[pl.ds(NS * SLICE, REM)],
                                  gsem[0]).wait()

        if with_deg:
            pltpu.sync_copy(zd_hbm.at[pl.ds(r0, SLICE)],
                            dacc.at[pl.ds(r0, SLICE)])

            @pl.when(s == 0)
            def _():
                pltpu.sync_copy(zd_hbm.at[pl.ds(NS * SLICE, REM)],
                                dacc.at[pl.ds(NS * SLICE, REM)])

            pltpu.sync_copy(ones_hbm, ones)
        plsc.subcore_barrier()

        e0 = wid * EDGES_PER_TILE

        def _sidx(ci):
            return src_all.at[pl.ds(ci * K, K)]

        def _didx(ci):
            return dst_all.at[pl.ds(ci * K, K)]

        def launch(ci, k):
            pltpu.async_copy(h_hbm.at[_sidx(ci)], bufs[k], gsem[k])

        def wait_gather(ci, k):
            pltpu.make_async_copy(h_hbm.at[_sidx(ci)], bufs[k],
                                  gsem[k]).wait()

        def start_scatter(ci, k):
            pltpu.async_copy(bufs[k], acc.at[_didx(ci)], ssem[k], add=True)
            if with_deg:
                pltpu.async_copy(ones, dacc.at[_didx(ci)], dsem[k], add=True)

        def wait_scatter(ci, k):
            pltpu.make_async_copy(bufs[k], acc.at[_didx(ci)], ssem[k]).wait()
            if with_deg:
                pltpu.make_async_copy(ones, dacc.at[_didx(ci)],
                                      dsem[k]).wait()

        def run_span(base, L):
            nw = L * K
            pltpu.sync_copy(src_hbm.at[pl.ds(e0 + base * K, nw)],
                            src_all.at[pl.ds(0, nw)])
            pltpu.sync_copy(dst_hbm.at[pl.ds(e0 + base * K, nw)],
                            dst_all.at[pl.ds(0, nw)])

            launch(0, 0)
            launch(1, 1)
            g = (L - 2) // 3

            def group(j, _):
                for k in range(3):
                    ci = 3 * j + k
                    ps = (k + 2) % 3
                    if k == 0:
                        @pl.when(j > 0)
                        def _():
                            wait_scatter(ci - 1, ps)
                    else:
                        wait_scatter(ci - 1, ps)
                    launch(ci + 2, ps)
                    wait_gather(ci, k)
                    start_scatter(ci, k)
                return ()

            lax.fori_loop(0, g, group, (), unroll=False)
            for ci in range(3 * g, L):
                k = ci % 3
                wait_scatter(ci - 1, (k + 2) % 3)
                if ci + 2 < L:
                    launch(ci + 2, (ci + 2) % 3)
                wait_gather(ci, k)
                start_scatter(ci, k)
            wait_scatter(L - 1, (L - 1) % 3)

        base = 0
        while base < CHUNKS:
            run_span(base, min(SPAN, CHUNKS - base))
            base += SPAN
        plsc.subcore_barrier()

        o0 = c * N + r0
        if d == 128:
            pltpu.sync_copy(acc.at[pl.ds(r0, SLICE)],
                            p_hbm.at[pl.ds(o0, SLICE)])
        else:
            pltpu.sync_copy(acc.at[pl.ds(r0, SLICE)],
                            p_hbm.at[pl.ds(o0, SLICE), pl.ds(0, d)])

        @pl.when(s == 0)
        def _():
            if d == 128:
                pltpu.sync_copy(acc.at[pl.ds(NS * SLICE, REM)],
                                p_hbm.at[pl.ds(c * N + NS * SLICE, REM)])
            else:
                pltpu.sync_copy(acc.at[pl.ds(NS * SLICE, REM)],
                                p_hbm.at[pl.ds(c * N + NS * SLICE, REM),
                                         pl.ds(0, d)])

        if with_deg:
            pltpu.sync_copy(dacc.at[pl.ds(r0, SLICE)],
                            dp_hbm.at[pl.ds(o0, SLICE), pl.ds(0, DEGW)])

            @pl.when(s == 0)
            def _():
                pltpu.sync_copy(dacc.at[pl.ds(NS * SLICE, REM)],
                                dp_hbm.at[pl.ds(c * N + NS * SLICE, REM),
                                          pl.ds(0, DEGW)])

    if with_deg:
        def body_wrap(h, src, dst, zd, ones_in, p, dp, src_all, dst_all,
                      b0, b1, b2, acc, g0, g1, g2, s0, s1, s2, ones, dacc,
                      d0, d1, d2):
            body(h, src, dst, zd, ones_in, p, dp, src_all, dst_all,
                 (b0, b1, b2), acc, (g0, g1, g2), (s0, s1, s2), ones, dacc,
                 (d0, d1, d2))
    else:
        def body_wrap(h, src, dst, p, src_all, dst_all, b0, b1, b2, acc,
                      g0, g1, g2, s0, s1, s2):
            body(h, src, dst, None, None, p, None, src_all, dst_all,
                 (b0, b1, b2), acc, (g0, g1, g2), (s0, s1, s2), None, None,
                 None)

    return pl.kernel(body_wrap, out_type=tuple(out_type), mesh=mesh,
                     scratch_types=scratch,
                     compiler_params=pltpu.CompilerParams(
                         use_tc_tiling_on_sc=False))


_seg_sum_deg_128 = _make_seg_sum(128, True, 80)
_seg_sum_64 = _make_seg_sum(64, False, 80)


BLK = 2000


def _stem_body(x_ref, w_ref, b_ref, o_ref):
    o_ref[...] = _leaky(_matT(x_ref[...], w_ref[...]) + b_ref[...])


def _esplit_body(ei_ref, src_ref, dst_ref):
    src_ref[...] = ei_ref[0]
    dst_ref[...] = ei_ref[1]


def _self_mm_body(a_ref, w_ref, o_ref):
    o_ref[...] = _matT(a_ref[...], w_ref[...])


def _layer1_body(hs1_ref, p_ref, dp_ref, wn_ref, b_ref, wn2_ref,
                 o1_ref, hw2_ref):
    ssum = p_ref[0] + p_ref[1]
    deg = dp_ref[0, :, 0:1] + dp_ref[1, :, 0:1]
    hn = ssum / jnp.maximum(deg, 1.0)
    o1 = _leaky(hs1_ref[...] + _matT(hn, wn_ref[...]) + b_ref[...])
    o1_ref[...] = o1
    hw2_ref[...] = _matT(o1, wn2_ref[...])


def _layer2_body(os2_ref, q_ref, dp_ref, b_ref, o_ref):
    ssum = q_ref[0, :, 0:64] + q_ref[1, :, 0:64]
    deg = dp_ref[0, :, 0:1] + dp_ref[1, :, 0:1]
    t = os2_ref[...] + ssum / jnp.maximum(deg, 1.0) \
        + b_ref[...]
    z = t - jnp.max(t, axis=1, keepdims=True)
    o_ref[...] = z - jnp.log(jnp.sum(jnp.exp(z), axis=1, keepdims=True))


def _row_spec(w):
    return pl.BlockSpec((BLK, w), lambda i: (i, 0))


def _pair_spec(w):
    return pl.BlockSpec((2, BLK, w), lambda i: (0, i, 0))


def _full_spec(shape):
    nd = len(shape)
    return pl.BlockSpec(shape, lambda i: (0,) * nd)


def kernel(x, adj, edge_index, W_lin, b_lin, W_self1, W_neigh1, b1,
           W_self2, W_neigh2, b2):
    del adj
    src, dst = pl.pallas_call(
        _esplit_body,
        out_shape=(jax.ShapeDtypeStruct((E,), jnp.int32),
                   jax.ShapeDtypeStruct((E,), jnp.int32)),
    )(edge_index)
    zd = jnp.zeros((N, DEGW), jnp.float32)

    h = pl.pallas_call(
        _stem_body,
        grid=(N // BLK,),
        in_specs=[_row_spec(128), _full_spec((128, 128)),
                  _full_spec((1, 128))],
        out_specs=_row_spec(128),
        out_shape=jax.ShapeDtypeStruct((N, 128), jnp.float32),
    )(x, W_lin, b_lin.reshape(1, -1))

    ones_in = jnp.ones((80, DEGW), jnp.float32)
    p, dp = _seg_sum_deg_128(h, src, dst, zd, ones_in)

    hs1 = pl.pallas_call(
        _self_mm_body,
        grid=(N // BLK,),
        in_specs=[_row_spec(128), _full_spec((128, 128))],
        out_specs=_row_spec(128),
        out_shape=jax.ShapeDtypeStruct((N, 128), jnp.float32),
    )(h, W_self1)

    o1, hw2 = pl.pallas_call(
        _layer1_body,
        grid=(N // BLK,),
        in_specs=[_row_spec(128), _pair_spec(128), _pair_spec(128),
                  _full_spec((128, 128)),
                  _full_spec((1, 128)), _full_spec((64, 128))],
        out_specs=(_row_spec(128), _row_spec(64)),
        out_shape=(jax.ShapeDtypeStruct((N, 128), jnp.float32),
                   jax.ShapeDtypeStruct((N, 64), jnp.float32)),
    )(hs1, p.reshape(2, N, 128), dp.reshape(2, N, 128), W_neigh1,
      b1.reshape(1, -1), W_neigh2)

    (q,) = _seg_sum_64(hw2, src, dst)

    os2 = pl.pallas_call(
        _self_mm_body,
        grid=(N // BLK,),
        in_specs=[_row_spec(128), _full_spec((64, 128))],
        out_specs=_row_spec(64),
        out_shape=jax.ShapeDtypeStruct((N, 64), jnp.float32),
    )(o1, W_self2)

    out = pl.pallas_call(
        _layer2_body,
        grid=(N // BLK,),
        in_specs=[_row_spec(64), _pair_spec(128), _pair_spec(128),
                  _full_spec((1, 64))],
        out_specs=_row_spec(64),
        out_shape=jax.ShapeDtypeStruct((N, 64), jnp.float32),
    )(os2, q.reshape(2, N, 128), dp.reshape(2, N, 128),
      b2.reshape(1, -1))
    return out


# speedup vs baseline: 1.0347x; 1.0347x over previous
import jax
import jax.numpy as jnp
from jax import lax
from jax.experimental import pallas as pl
from jax.experimental.pallas import tpu as pltpu
from jax.experimental.pallas import tpu_sc as plsc

N = 10000
E = 320000
ALPHA = 0.2

NC = 2
NS = 16
NW = NC * NS
EDGES_PER_TILE = E // NW
SLICE = 624
REM = N - NS * SLICE
DEGW = 8


def _leaky(v):
    return jnp.where(v >= 0, v, ALPHA * v)


def _matT(a, w):
    return lax.dot_general(a, w, (((1,), (1,)), ((), ())),
                           preferred_element_type=jnp.float32)


def _make_seg_sum(d, with_deg, K):
    CHUNKS = EDGES_PER_TILE // K
    TAIL = EDGES_PER_TILE - CHUNKS * K
    assert K % 8 == 0 and TAIL % 8 == 0
    SPAN = (CHUNKS + 1) // 2
    IDXW = SPAN * K + TAIL
    mesh = plsc.VectorSubcoreMesh(core_axis_name="c", subcore_axis_name="s")
    out_type = [jax.ShapeDtypeStruct((NC * N, 128), jnp.float32)]
    scratch = [
        pltpu.VMEM((IDXW,), jnp.int32),
        pltpu.VMEM((IDXW,), jnp.int32),
        pltpu.VMEM((K, d), jnp.float32),
        pltpu.VMEM((K, d), jnp.float32),
        pltpu.VMEM((K, d), jnp.float32),
        pltpu.VMEM_SHARED((N, d), jnp.float32),
        pltpu.SemaphoreType.DMA,
        pltpu.SemaphoreType.DMA,
        pltpu.SemaphoreType.DMA,
        pltpu.SemaphoreType.DMA,
        pltpu.SemaphoreType.DMA,
        pltpu.SemaphoreType.DMA,
    ]
    if with_deg:
        out_type.append(jax.ShapeDtypeStruct((NC * N, 128), jnp.float32))
        scratch += [
            pltpu.VMEM((K, DEGW), jnp.float32),
            pltpu.VMEM_SHARED((N, DEGW), jnp.float32),
            pltpu.SemaphoreType.DMA,
            pltpu.SemaphoreType.DMA,
            pltpu.SemaphoreType.DMA,
        ]

    def body(h_hbm, src_hbm, dst_hbm, zd_hbm, ones_hbm, p_hbm,
             dp_hbm, src_all, dst_all, bufs, acc, gsem, ssem, ones, dacc,
             dsem):
        c = lax.axis_index("c")
        s = lax.axis_index("s")
        wid = c * NS + s
        r0 = s * SLICE

        def zrow(i, _):
            for jj in range(d // 16):
                bufs[0][i, pl.ds(16 * jj, 16)] = jnp.zeros((16,), jnp.float32)
            return ()

        lax.fori_loop(0, K, zrow, (), unroll=False)
        for t in range(SLICE // K):
            pltpu.async_copy(bufs[0], acc.at[pl.ds(r0 + t * K, K)], gsem[0])
        zrem = SLICE - (SLICE // K) * K
        if zrem:
            pltpu.async_copy(bufs[0].at[pl.ds(0, zrem)],
                             acc.at[pl.ds(r0 + SLICE - zrem, zrem)], gsem[0])

        @pl.when(s == 0)
        def _():
            pltpu.async_copy(bufs[0].at[pl.ds(0, REM)],
                             acc.at[pl.ds(NS * SLICE, REM)], gsem[0])
        for t in range(SLICE // K):
            pltpu.make_async_copy(bufs[0], acc.at[pl.ds(r0 + t * K, K)],
                                  gsem[0]).wait()
        if zrem:
            pltpu.make_async_copy(bufs[0].at[pl.ds(0, zrem)],
                                  acc.at[pl.ds(r0 + SLICE - zrem, zrem)],
                                  gsem[0]).wait()

        @pl.when(s == 0)
        def _():
            pltpu.make_async_copy(bufs[0].at[pl.ds(0, REM)],
                                  acc.at[pl.ds(NS * SLICE, REM)],
                                  gsem[0]).wait()

        if with_deg:
            pltpu.sync_copy(zd_hbm.at[pl.ds(r0, SLICE)],
                            dacc.at[pl.ds(r0, SLICE)])

            @pl.when(s == 0)
            def _():
                pltpu.sync_copy(zd_hbm.at[pl.ds(NS * SLICE, REM)],
                                dacc.at[pl.ds(NS * SLICE, REM)])

            pltpu.sync_copy(ones_hbm, ones)
        plsc.subcore_barrier()

        e0 = wid * EDGES_PER_TILE

        def _sidx(ci):
            return src_all.at[pl.ds(ci * K, K)]

        def _didx(ci):
            return dst_all.at[pl.ds(ci * K, K)]

        def launch(ci, k):
            pltpu.async_copy(h_hbm.at[_sidx(ci)], bufs[k], gsem[k])

        def wait_gather(ci, k):
            pltpu.make_async_copy(h_hbm.at[_sidx(ci)], bufs[k],
                                  gsem[k]).wait()

        def start_scatter(ci, k):
            pltpu.async_copy(bufs[k], acc.at[_didx(ci)], ssem[k], add=True)
            if with_deg:
                pltpu.async_copy(ones, dacc.at[_didx(ci)], dsem[k], add=True)

        def wait_scatter(ci, k):
            pltpu.make_async_copy(bufs[k], acc.at[_didx(ci)], ssem[k]).wait()
            if with_deg:
                pltpu.make_async_copy(ones, dacc.at[_didx(ci)],
                                      dsem[k]).wait()

        def run_span(base, L, extra):
            nw = L * K + extra
            pltpu.sync_copy(src_hbm.at[pl.ds(e0 + base * K, nw)],
                            src_all.at[pl.ds(0, nw)])
            pltpu.sync_copy(dst_hbm.at[pl.ds(e0 + base * K, nw)],
                            dst_all.at[pl.ds(0, nw)])

            launch(0, 0)
            launch(1, 1)
            g = (L - 2) // 3

            def group(j, _):
                for k in range(3):
                    ci = 3 * j + k
                    ps = (k + 2) % 3
                    if k == 0:
                        @pl.when(j > 0)
                        def _():
                            wait_scatter(ci - 1, ps)
                    else:
                        wait_scatter(ci - 1, ps)
                    launch(ci + 2, ps)
                    wait_gather(ci, k)
                    start_scatter(ci, k)
                return ()

            lax.fori_loop(0, g, group, (), unroll=False)
            for ci in range(3 * g, L):
                k = ci % 3
                wait_scatter(ci - 1, (k + 2) % 3)
                if ci + 2 < L:
                    launch(ci + 2, (ci + 2) % 3)
                wait_gather(ci, k)
                start_scatter(ci, k)
            wait_scatter(L - 1, (L - 1) % 3)

        base = 0
        while base < CHUNKS:
            L = min(SPAN, CHUNKS - base)
            run_span(base, L, TAIL if base + L == CHUNKS else 0)
            base += L
        if TAIL:
            ti = (CHUNKS - (CHUNKS - SPAN)) * K if CHUNKS > SPAN else \
                CHUNKS * K
            tsrc = src_all.at[pl.ds(ti, TAIL)]
            tdst = dst_all.at[pl.ds(ti, TAIL)]
            tbuf = bufs[2].at[pl.ds(0, TAIL)]
            pltpu.async_copy(h_hbm.at[tsrc], tbuf, gsem[2])
            pltpu.make_async_copy(h_hbm.at[tsrc], tbuf, gsem[2]).wait()
            pltpu.async_copy(tbuf, acc.at[tdst], ssem[2], add=True)
            pltpu.make_async_copy(tbuf, acc.at[tdst], ssem[2]).wait()
        plsc.subcore_barrier()

        o0 = c * N + r0
        if d == 128:
            pltpu.sync_copy(acc.at[pl.ds(r0, SLICE)],
                            p_hbm.at[pl.ds(o0, SLICE)])
        else:
            pltpu.sync_copy(acc.at[pl.ds(r0, SLICE)],
                            p_hbm.at[pl.ds(o0, SLICE), pl.ds(0, d)])

        @pl.when(s == 0)
        def _():
            if d == 128:
                pltpu.sync_copy(acc.at[pl.ds(NS * SLICE, REM)],
                                p_hbm.at[pl.ds(c * N + NS * SLICE, REM)])
            else:
                pltpu.sync_copy(acc.at[pl.ds(NS * SLICE, REM)],
                                p_hbm.at[pl.ds(c * N + NS * SLICE, REM),
                                         pl.ds(0, d)])

        if with_deg:
            pltpu.sync_copy(dacc.at[pl.ds(r0, SLICE)],
                            dp_hbm.at[pl.ds(o0, SLICE), pl.ds(0, DEGW)])

            @pl.when(s == 0)
            def _():
                pltpu.sync_copy(dacc.at[pl.ds(NS * SLICE, REM)],
                                dp_hbm.at[pl.ds(c * N + NS * SLICE, REM),
                                          pl.ds(0, DEGW)])

    if with_deg:
        def body_wrap(h, src, dst, zd, ones_in, p, dp, src_all, dst_all,
                      b0, b1, b2, acc, g0, g1, g2, s0, s1, s2, ones, dacc,
                      d0, d1, d2):
            body(h, src, dst, zd, ones_in, p, dp, src_all, dst_all,
                 (b0, b1, b2), acc, (g0, g1, g2), (s0, s1, s2), ones, dacc,
                 (d0, d1, d2))
    else:
        def body_wrap(h, src, dst, p, src_all, dst_all, b0, b1, b2, acc,
                      g0, g1, g2, s0, s1, s2):
            body(h, src, dst, None, None, p, None, src_all, dst_all,
                 (b0, b1, b2), acc, (g0, g1, g2), (s0, s1, s2), None, None,
                 None)

    return pl.kernel(body_wrap, out_type=tuple(out_type), mesh=mesh,
                     scratch_types=scratch,
                     compiler_params=pltpu.CompilerParams(
                         use_tc_tiling_on_sc=False))


_seg_sum_deg_128 = _make_seg_sum(128, True, 80)
_seg_sum_64 = _make_seg_sum(64, False, 128)


BLK = 2000


def _stem_body(x_ref, w_ref, b_ref, o_ref):
    o_ref[...] = _leaky(_matT(x_ref[...], w_ref[...]) + b_ref[...])


def _esplit_body(ei_ref, src_ref, dst_ref):
    src_ref[...] = ei_ref[0]
    dst_ref[...] = ei_ref[1]


def _layer1_body(h_ref, p_ref, dp_ref, ws_ref, wn_ref, b_ref, wn2_ref,
                 o1_ref, hw2_ref):
    ssum = p_ref[0] + p_ref[1]
    deg = dp_ref[0, :, 0:1] + dp_ref[1, :, 0:1]
    hn = ssum / jnp.maximum(deg, 1.0)
    o1 = _leaky(_matT(h_ref[...], ws_ref[...]) + _matT(hn, wn_ref[...])
                + b_ref[...])
    o1_ref[...] = o1
    hw2_ref[...] = _matT(o1, wn2_ref[...])


def _layer2_body(o1_ref, q_ref, dp_ref, ws_ref, b_ref, o_ref):
    ssum = q_ref[0, :, 0:64] + q_ref[1, :, 0:64]
    deg = dp_ref[0, :, 0:1] + dp_ref[1, :, 0:1]
    t = _matT(o1_ref[...], ws_ref[...]) + ssum / jnp.maximum(deg, 1.0) \
        + b_ref[...]
    z = t - jnp.max(t, axis=1, keepdims=True)
    o_ref[...] = z - jnp.log(jnp.sum(jnp.exp(z), axis=1, keepdims=True))


def _row_spec(w):
    return pl.BlockSpec((BLK, w), lambda i: (i, 0))


def _pair_spec(w):
    return pl.BlockSpec((2, BLK, w), lambda i: (0, i, 0))


def _full_spec(shape):
    nd = len(shape)
    return pl.BlockSpec(shape, lambda i: (0,) * nd)


def kernel(x, adj, edge_index, W_lin, b_lin, W_self1, W_neigh1, b1,
           W_self2, W_neigh2, b2):
    del adj
    src, dst = pl.pallas_call(
        _esplit_body,
        out_shape=(jax.ShapeDtypeStruct((E,), jnp.int32),
                   jax.ShapeDtypeStruct((E,), jnp.int32)),
    )(edge_index)
    zd = jnp.zeros((N, DEGW), jnp.float32)

    h = pl.pallas_call(
        _stem_body,
        grid=(N // BLK,),
        in_specs=[_row_spec(128), _full_spec((128, 128)),
                  _full_spec((1, 128))],
        out_specs=_row_spec(128),
        out_shape=jax.ShapeDtypeStruct((N, 128), jnp.float32),
    )(x, W_lin, b_lin.reshape(1, -1))

    ones_in = jnp.ones((80, DEGW), jnp.float32)
    p, dp = _seg_sum_deg_128(h, src, dst, zd, ones_in)

    o1, hw2 = pl.pallas_call(
        _layer1_body,
        grid=(N // BLK,),
        in_specs=[_row_spec(128), _pair_spec(128), _pair_spec(128),
                  _full_spec((128, 128)), _full_spec((128, 128)),
                  _full_spec((1, 128)), _full_spec((64, 128))],
        out_specs=(_row_spec(128), _row_spec(64)),
        out_shape=(jax.ShapeDtypeStruct((N, 128), jnp.float32),
                   jax.ShapeDtypeStruct((N, 64), jnp.float32)),
    )(h, p.reshape(2, N, 128), dp.reshape(2, N, 128), W_self1, W_neigh1,
      b1.reshape(1, -1), W_neigh2)

    (q,) = _seg_sum_64(hw2, src, dst)

    out = pl.pallas_call(
        _layer2_body,
        grid=(N // BLK,),
        in_specs=[_row_spec(128), _pair_spec(128), _pair_spec(128),
                  _full_spec((64, 128)), _full_spec((1, 64))],
        out_specs=_row_spec(64),
        out_shape=jax.ShapeDtypeStruct((N, 64), jnp.float32),
    )(o1, q.reshape(2, N, 128), dp.reshape(2, N, 128), W_self2,
      b2.reshape(1, -1))
    return out
